# Initial kernel scaffold; baseline (speedup 1.0000x reference)
import jax
import jax.numpy as jnp
from jax.experimental import pallas as pl


def kernel(pd_scores, pd_bboxes, anc_points, gt_labels, gt_bboxes):
    raise NotImplementedError("write your pallas kernel here")


# speedup vs baseline: 20.9266x; 20.9266x over previous
import functools
import math

import jax
import jax.numpy as jnp
from jax.experimental import pallas as pl

_NUM_CLASSES = 80
_IOU_WEIGHT = 3.0
_LAMBDA_BOX = 7.5
_LAMBDA_CLS = 0.5
_I32_MIN = -(2**31)
_I32_MAX = 2**31 - 1


def _sortable_key(cost):
    bits = jax.lax.bitcast_convert_type(cost, jnp.int32)
    return bits ^ (jnp.right_shift(bits, 31) & 0x7FFFFFFF)


def _arctan(x):
    s = jnp.sign(x)
    ax = jnp.abs(x)
    inv = ax > 1.0
    t = jnp.where(inv, 1.0 / jnp.maximum(ax, 1e-30), ax)
    t2 = t * t
    p = -0.0040540580
    for coef in (0.0218612288, -0.0559098861, 0.0964200441, -0.1390853351,
                 0.1994653599, -0.3332985605, 0.9999993329):
        p = p * t2 + coef
    r = t * p
    return s * jnp.where(inv, math.pi / 2 - r, r)


def _softplus(x):
    return jnp.maximum(x, 0.0) + jnp.log1p(jnp.exp(-jnp.abs(x)))


def _iou_and_mask(gtb_ref, pdbT_ref, ancT_ref):
    gx1 = gtb_ref[:, 0:1]
    gy1 = gtb_ref[:, 1:2]
    gx2 = gtb_ref[:, 2:3]
    gy2 = gtb_ref[:, 3:4]
    px1 = pdbT_ref[0:1, :]
    py1 = pdbT_ref[1:2, :]
    px2 = pdbT_ref[2:3, :]
    py2 = pdbT_ref[3:4, :]
    ax = ancT_ref[0:1, :]
    ay = ancT_ref[1:2, :]
    ix1 = jnp.maximum(gx1, px1)
    iy1 = jnp.maximum(gy1, py1)
    ix2 = jnp.minimum(gx2, px2)
    iy2 = jnp.minimum(gy2, py2)
    inter = jnp.clip(ix2 - ix1, 0, None) * jnp.clip(iy2 - iy1, 0, None)
    a1 = (gx2 - gx1) * (gy2 - gy1)
    a2 = (px2 - px1) * (py2 - py1)
    iou = inter / (a1 + a2 - inter + 1e-9)
    is_in = jnp.minimum(jnp.minimum(ax - gx1, ay - gy1),
                        jnp.minimum(gx2 - ax, gy2 - ay)) > 0.01
    return iou, is_in


def _pps(cls_ref, scores):
    c = scores.shape[-1]
    iota_c = jax.lax.broadcasted_iota(jnp.int32, (1, c), 1)
    onehot = (cls_ref[...] == iota_c).astype(jnp.float32)
    return jax.lax.dot_general(
        onehot, scores, (((1,), (1,)), ((), ())),
        precision=jax.lax.Precision.HIGHEST,
        preferred_element_type=jnp.float32)


def _phase1(na, scores_ref, pdbT_ref, ancT_ref, gtb_ref, cls_ref,
            key_ref, sumiou_ref):
    j = pl.program_id(0)
    iou, is_in = _iou_and_mask(gtb_ref, pdbT_ref, ancT_ref)
    pps = _pps(cls_ref, scores_ref[...])
    cost = (_softplus(-pps) - _IOU_WEIGHT * jnp.log(iou + 1e-8)
            + 1e5 * (~is_in).astype(jnp.float32))
    g, ab = cost.shape
    idx = j * ab + jax.lax.broadcasted_iota(jnp.int32, (g, ab), 1)
    key_ref[...] = jnp.where(idx < na, _sortable_key(cost), _I32_MAX)

    @pl.when(j == 0)
    def _():
        sumiou_ref[...] = jnp.zeros_like(sumiou_ref)

    iou_m = jnp.where(is_in, iou, 0.0)
    sumiou_ref[...] += iou_m.sum(axis=1, keepdims=True)


def _phase2(key_ref, sumiou_ref, t_ref, i_ref):
    key = key_ref[...]
    g, a = key.shape
    k = jnp.clip(sumiou_ref[...].astype(jnp.int32), 1, a)

    def body_val(_, lh):
        lo, hi = lh
        mid = lo + jax.lax.shift_right_logical(hi - lo, 1)
        cnt = (key <= mid).astype(jnp.int32).sum(axis=1, keepdims=True)
        pred = cnt >= k
        return (jnp.where(pred, lo, mid + 1), jnp.where(pred, mid, hi))

    lo, _ = jax.lax.fori_loop(
        0, 32, body_val,
        (jnp.full((g, 1), _I32_MIN, jnp.int32),
         jnp.full((g, 1), _I32_MAX, jnp.int32)))
    t = lo
    n_lt = (key < t).astype(jnp.int32).sum(axis=1, keepdims=True)
    need = k - n_lt
    eq = key == t
    idx = jax.lax.broadcasted_iota(jnp.int32, (1, a), 1)

    def body_idx(_, lh):
        lo, hi = lh
        mid = lo + jax.lax.shift_right_logical(hi - lo, 1)
        cnt = (eq & (idx <= mid)).astype(jnp.int32).sum(axis=1, keepdims=True)
        pred = cnt >= need
        return (jnp.where(pred, lo, mid + 1), jnp.where(pred, mid, hi))

    lo2, _ = jax.lax.fori_loop(
        0, 15, body_idx,
        (jnp.zeros((g, 1), jnp.int32), jnp.full((g, 1), a - 1, jnp.int32)))
    t_ref[...] = t
    i_ref[...] = lo2


def _phase3(key_ref, scores_ref, pdbT_ref, ancT_ref, gtb_ref, cls_ref,
            t_ref, i_ref, sp_ref, dot_ref, s1_ref, box_ref):
    j = pl.program_id(0)
    key = key_ref[...]
    g, ab = key.shape
    idx = j * ab + jax.lax.broadcasted_iota(jnp.int32, (g, ab), 1)
    mm = (key < t_ref[...]) | ((key == t_ref[...]) & (idx <= i_ref[...]))
    mmf = mm.astype(jnp.float32)
    amg = mmf.sum(axis=0, keepdims=True)
    conflict_f = (amg > 1.0).astype(jnp.float32)
    minkey = key.min(axis=0, keepdims=True)
    iota_g = jax.lax.broadcasted_iota(jnp.int32, (g, ab), 0)
    cm = jnp.where(key == minkey, iota_g, jnp.int32(g)).min(
        axis=0, keepdims=True)
    onehot_f = (iota_g == cm).astype(jnp.float32)
    mmf2 = conflict_f * onehot_f + (1.0 - conflict_f) * mmf

    iou, is_in = _iou_and_mask(gtb_ref, pdbT_ref, ancT_ref)
    iou_m = jnp.where(is_in, iou, 0.0)
    matched_iou = (mmf2 * iou_m).sum(axis=0, keepdims=True)
    pps = _pps(cls_ref, scores_ref[...])
    dot_blk = (mmf2 * iou_m * pps).sum(keepdims=True).reshape(1, 1)
    sp_blk = _softplus(scores_ref[...]).sum(keepdims=True).reshape(1, 1)
    s1_blk = matched_iou.sum(keepdims=True).reshape(1, 1)

    bx1 = (mmf2 * gtb_ref[:, 0:1]).sum(axis=0, keepdims=True)
    by1 = (mmf2 * gtb_ref[:, 1:2]).sum(axis=0, keepdims=True)
    bx2 = (mmf2 * gtb_ref[:, 2:3]).sum(axis=0, keepdims=True)
    by2 = (mmf2 * gtb_ref[:, 3:4]).sum(axis=0, keepdims=True)
    px1 = pdbT_ref[0:1, :]
    py1 = pdbT_ref[1:2, :]
    px2 = pdbT_ref[2:3, :]
    py2 = pdbT_ref[3:4, :]
    eps = 1e-7
    w1 = px2 - px1
    h1 = py2 - py1 + eps
    w2 = bx2 - bx1
    h2 = by2 - by1 + eps
    inter = (jnp.clip(jnp.minimum(px2, bx2) - jnp.maximum(px1, bx1), 0, None)
             * jnp.clip(jnp.minimum(py2, by2) - jnp.maximum(py1, by1), 0, None))
    union = w1 * h1 + w2 * h2 - inter + eps
    iou2 = inter / union
    cw = jnp.maximum(px2, bx2) - jnp.minimum(px1, bx1)
    ch = jnp.maximum(py2, by2) - jnp.minimum(py1, by1)
    c2 = cw * cw + ch * ch + eps
    rho2 = ((bx1 + bx2 - px1 - px2) ** 2 + (by1 + by2 - py1 - py2) ** 2) / 4.0
    v = (4.0 / math.pi**2) * (_arctan(w2 / h2) - _arctan(w1 / h1)) ** 2
    alpha = v / (v - iou2 + (1.0 + eps))
    ciou = iou2 - (rho2 / c2 + v * alpha)
    box_blk = ((1.0 - ciou) * matched_iou).sum(keepdims=True).reshape(1, 1)

    @pl.when(j == 0)
    def _():
        sp_ref[...] = jnp.zeros_like(sp_ref)
        dot_ref[...] = jnp.zeros_like(dot_ref)
        s1_ref[...] = jnp.zeros_like(s1_ref)
        box_ref[...] = jnp.zeros_like(box_ref)

    sp_ref[...] += sp_blk
    dot_ref[...] += dot_blk
    s1_ref[...] += s1_blk
    box_ref[...] += box_blk


def kernel(pd_scores, pd_bboxes, anc_points, gt_labels, gt_bboxes):
    a, c = pd_scores.shape
    g = gt_bboxes.shape[0]
    nb = 4
    ap = -(-a // (nb * 128)) * nb * 128
    pad = ap - a
    ab = ap // nb
    pd_scores = jnp.pad(pd_scores, ((0, pad), (0, 0)), constant_values=-1e4)
    pdbT = jnp.pad(pd_bboxes.T, ((0, 0), (0, pad)))
    ancT = jnp.pad(anc_points.T, ((0, 0), (0, pad)), constant_values=-1e6)
    cls2d = gt_labels[:, 0:1].astype(jnp.int32)
    key, sumiou = pl.pallas_call(
        functools.partial(_phase1, a),
        grid=(nb,),
        in_specs=[
            pl.BlockSpec((ab, c), lambda j: (j, 0)),
            pl.BlockSpec((4, ab), lambda j: (0, j)),
            pl.BlockSpec((2, ab), lambda j: (0, j)),
            pl.BlockSpec((g, 4), lambda j: (0, 0)),
            pl.BlockSpec((g, 1), lambda j: (0, 0)),
        ],
        out_specs=[
            pl.BlockSpec((g, ab), lambda j: (0, j)),
            pl.BlockSpec((g, 1), lambda j: (0, 0)),
        ],
        out_shape=[
            jax.ShapeDtypeStruct((g, ap), jnp.int32),
            jax.ShapeDtypeStruct((g, 1), jnp.float32),
        ],
    )(pd_scores, pdbT, ancT, gt_bboxes, cls2d)

    t, tie_i = pl.pallas_call(
        _phase2,
        out_shape=[
            jax.ShapeDtypeStruct((g, 1), jnp.int32),
            jax.ShapeDtypeStruct((g, 1), jnp.int32),
        ],
    )(key, sumiou)

    sp, dot, s1, box = pl.pallas_call(
        _phase3,
        grid=(nb,),
        in_specs=[
            pl.BlockSpec((g, ab), lambda j: (0, j)),
            pl.BlockSpec((ab, c), lambda j: (j, 0)),
            pl.BlockSpec((4, ab), lambda j: (0, j)),
            pl.BlockSpec((2, ab), lambda j: (0, j)),
            pl.BlockSpec((g, 4), lambda j: (0, 0)),
            pl.BlockSpec((g, 1), lambda j: (0, 0)),
            pl.BlockSpec((g, 1), lambda j: (0, 0)),
            pl.BlockSpec((g, 1), lambda j: (0, 0)),
        ],
        out_specs=[pl.BlockSpec((1, 1), lambda j: (0, 0))] * 4,
        out_shape=[jax.ShapeDtypeStruct((1, 1), jnp.float32)] * 4,
    )(key, pd_scores, pdbT, ancT, gt_bboxes, cls2d, t, tie_i)

    tss = jnp.maximum(s1[0, 0], 1.0)
    return (_LAMBDA_BOX * box[0, 0]
            + _LAMBDA_CLS * (sp[0, 0] - dot[0, 0])) / tss


# speedup vs baseline: 30.6394x; 1.4641x over previous
import math

import jax
import jax.numpy as jnp
from jax.experimental import pallas as pl

_NUM_CLASSES = 80
_IOU_WEIGHT = 3.0
_LAMBDA_BOX = 7.5
_LAMBDA_CLS = 0.5
_I32_MIN = -(2**31)
_I32_MAX = 2**31 - 1


def _sortable_key(cost):
    bits = jax.lax.bitcast_convert_type(cost, jnp.int32)
    return bits ^ (jnp.right_shift(bits, 31) & 0x7FFFFFFF)


def _arctan(x):
    s = jnp.sign(x)
    ax = jnp.abs(x)
    inv = ax > 1.0
    t = jnp.where(inv, 1.0 / jnp.maximum(ax, 1e-30), ax)
    t2 = t * t
    p = -0.0040540580
    for coef in (0.0218612288, -0.0559098861, 0.0964200441, -0.1390853351,
                 0.1994653599, -0.3332985605, 0.9999993329):
        p = p * t2 + coef
    r = t * p
    return s * jnp.where(inv, math.pi / 2 - r, r)


def _softplus(x):
    return jnp.maximum(x, 0.0) + jnp.log1p(jnp.exp(-jnp.abs(x)))


def _fused(scores_ref, pdbT_ref, ancT_ref, gtb_ref, cls_ref, loss_ref):
    g = gtb_ref.shape[0]
    a, c = scores_ref.shape

    gx1 = gtb_ref[:, 0:1]
    gy1 = gtb_ref[:, 1:2]
    gx2 = gtb_ref[:, 2:3]
    gy2 = gtb_ref[:, 3:4]
    px1 = pdbT_ref[0:1, :]
    py1 = pdbT_ref[1:2, :]
    px2 = pdbT_ref[2:3, :]
    py2 = pdbT_ref[3:4, :]
    ax = ancT_ref[0:1, :]
    ay = ancT_ref[1:2, :]
    ix1 = jnp.maximum(gx1, px1)
    iy1 = jnp.maximum(gy1, py1)
    ix2 = jnp.minimum(gx2, px2)
    iy2 = jnp.minimum(gy2, py2)
    inter = jnp.clip(ix2 - ix1, 0, None) * jnp.clip(iy2 - iy1, 0, None)
    a1 = (gx2 - gx1) * (gy2 - gy1)
    a2 = (px2 - px1) * (py2 - py1)
    iou = inter / (a1 + a2 - inter + 1e-9)
    is_in = jnp.minimum(jnp.minimum(ax - gx1, ay - gy1),
                        jnp.minimum(gx2 - ax, gy2 - ay)) > 0.01
    iota_c = jax.lax.broadcasted_iota(jnp.int32, (1, c), 1)
    onehot = (cls_ref[...] == iota_c).astype(jnp.float32)
    pps = jax.lax.dot_general(
        onehot, scores_ref[...], (((1,), (1,)), ((), ())),
        precision=jax.lax.Precision.HIGHEST,
        preferred_element_type=jnp.float32)
    cost = (_softplus(-pps) - _IOU_WEIGHT * jnp.log(iou + 1e-8)
            + 1e5 * (~is_in).astype(jnp.float32))
    key = _sortable_key(cost)
    iou_m = jnp.where(is_in, iou, 0.0)

    k = jnp.clip(iou_m.sum(axis=1, keepdims=True).astype(jnp.int32), 1, a)

    def body_val(_, lh):
        lo, hi = lh
        mid = lo + jax.lax.shift_right_logical(hi - lo, 1)
        cnt = (key <= mid).astype(jnp.int32).sum(axis=1, keepdims=True)
        pred = cnt >= k
        return (jnp.where(pred, lo, mid + 1), jnp.where(pred, mid, hi))

    t, _ = jax.lax.fori_loop(
        0, 32, body_val,
        (jnp.full((g, 1), _I32_MIN, jnp.int32),
         jnp.full((g, 1), _I32_MAX, jnp.int32)))
    n_lt = (key < t).astype(jnp.int32).sum(axis=1, keepdims=True)
    need = k - n_lt

    def body_idx(_, lh):
        lo, hi = lh
        mid = lo + jax.lax.shift_right_logical(hi - lo, 1)
        idx_b = jax.lax.broadcasted_iota(jnp.int32, (g, a), 1)
        cnt = ((key == t) & (idx_b <= mid)).astype(jnp.int32).sum(
            axis=1, keepdims=True)
        pred = cnt >= need
        return (jnp.where(pred, lo, mid + 1), jnp.where(pred, mid, hi))

    tie_i, _ = jax.lax.fori_loop(
        0, 15, body_idx,
        (jnp.zeros((g, 1), jnp.int32), jnp.full((g, 1), a - 1, jnp.int32)))

    idx = jax.lax.broadcasted_iota(jnp.int32, (g, a), 1)
    mm = (key < t) | ((key == t) & (idx <= tie_i))
    mmf = mm.astype(jnp.float32)
    amg = mmf.sum(axis=0, keepdims=True)
    conflict_f = (amg > 1.0).astype(jnp.float32)
    minkey = key.min(axis=0, keepdims=True)
    iota_g = jax.lax.broadcasted_iota(jnp.int32, (g, a), 0)
    cm = jnp.where(key == minkey, iota_g, jnp.int32(g)).min(
        axis=0, keepdims=True)
    onehot_f = (iota_g == cm).astype(jnp.float32)
    mmf2 = conflict_f * onehot_f + (1.0 - conflict_f) * mmf

    matched_iou = (mmf2 * iou_m).sum(axis=0, keepdims=True)
    dot_s = (mmf2 * iou_m * pps).sum()
    sp_s = _softplus(scores_ref[...]).sum()
    s1 = matched_iou.sum()

    bx1 = (mmf2 * gx1).sum(axis=0, keepdims=True)
    by1 = (mmf2 * gy1).sum(axis=0, keepdims=True)
    bx2 = (mmf2 * gx2).sum(axis=0, keepdims=True)
    by2 = (mmf2 * gy2).sum(axis=0, keepdims=True)
    eps = 1e-7
    w1 = px2 - px1
    h1 = py2 - py1 + eps
    w2 = bx2 - bx1
    h2 = by2 - by1 + eps
    inter2 = (jnp.clip(jnp.minimum(px2, bx2) - jnp.maximum(px1, bx1), 0, None)
              * jnp.clip(jnp.minimum(py2, by2) - jnp.maximum(py1, by1), 0, None))
    union2 = w1 * h1 + w2 * h2 - inter2 + eps
    iou2 = inter2 / union2
    cw = jnp.maximum(px2, bx2) - jnp.minimum(px1, bx1)
    ch = jnp.maximum(py2, by2) - jnp.minimum(py1, by1)
    c2 = cw * cw + ch * ch + eps
    rho2 = ((bx1 + bx2 - px1 - px2) ** 2 + (by1 + by2 - py1 - py2) ** 2) / 4.0
    v = (4.0 / math.pi**2) * (_arctan(w2 / h2) - _arctan(w1 / h1)) ** 2
    alpha = v / (v - iou2 + (1.0 + eps))
    ciou = iou2 - (rho2 / c2 + v * alpha)
    box_s = ((1.0 - ciou) * matched_iou).sum()

    tss = jnp.maximum(s1, 1.0)
    loss = (_LAMBDA_BOX * box_s + _LAMBDA_CLS * (sp_s - dot_s)) / tss
    loss_ref[...] = loss.reshape(1, 1)


def kernel(pd_scores, pd_bboxes, anc_points, gt_labels, gt_bboxes):
    cls2d = gt_labels[:, 0:1].astype(jnp.int32)
    out = pl.pallas_call(
        _fused,
        out_shape=jax.ShapeDtypeStruct((1, 1), jnp.float32),
    )(pd_scores, pd_bboxes.T, anc_points.T, gt_bboxes, cls2d)
    return out[0, 0]


# speedup vs baseline: 35.9052x; 1.1719x over previous
import math

import jax
import jax.numpy as jnp
from jax.experimental import pallas as pl

_NUM_CLASSES = 80
_IOU_WEIGHT = 3.0
_LAMBDA_BOX = 7.5
_LAMBDA_CLS = 0.5
_I32_MIN = -(2**31)
_I32_MAX = 2**31 - 1


def _sortable_key(cost):
    bits = jax.lax.bitcast_convert_type(cost, jnp.int32)
    return bits ^ (jnp.right_shift(bits, 31) & 0x7FFFFFFF)


def _arctan(x):
    s = jnp.sign(x)
    ax = jnp.abs(x)
    inv = ax > 1.0
    t = jnp.where(inv, 1.0 / jnp.maximum(ax, 1e-30), ax)
    t2 = t * t
    p = -0.0040540580
    for coef in (0.0218612288, -0.0559098861, 0.0964200441, -0.1390853351,
                 0.1994653599, -0.3332985605, 0.9999993329):
        p = p * t2 + coef
    r = t * p
    return s * jnp.where(inv, math.pi / 2 - r, r)


def _softplus(x):
    return jnp.maximum(x, 0.0) + jnp.log1p(jnp.exp(-jnp.abs(x)))


def _rowcount(mask):
    g, a = mask.shape
    step = 2560
    parts = [mask[:, o:min(o + step, a)].astype(jnp.int32).sum(
        axis=1, keepdims=True) for o in range(0, a, step)]
    while len(parts) > 1:
        parts = [parts[i] + parts[i + 1] if i + 1 < len(parts) else parts[i]
                 for i in range(0, len(parts), 2)]
    return parts[0]


def _fused(scores_ref, pdbT_ref, ancT_ref, gtb_ref, cls_ref, loss_ref):
    g = gtb_ref.shape[0]
    a, c = scores_ref.shape

    gx1 = gtb_ref[:, 0:1]
    gy1 = gtb_ref[:, 1:2]
    gx2 = gtb_ref[:, 2:3]
    gy2 = gtb_ref[:, 3:4]
    px1 = pdbT_ref[0:1, :]
    py1 = pdbT_ref[1:2, :]
    px2 = pdbT_ref[2:3, :]
    py2 = pdbT_ref[3:4, :]
    ax = ancT_ref[0:1, :]
    ay = ancT_ref[1:2, :]
    ix1 = jnp.maximum(gx1, px1)
    iy1 = jnp.maximum(gy1, py1)
    ix2 = jnp.minimum(gx2, px2)
    iy2 = jnp.minimum(gy2, py2)
    inter = jnp.clip(ix2 - ix1, 0, None) * jnp.clip(iy2 - iy1, 0, None)
    a1 = (gx2 - gx1) * (gy2 - gy1)
    a2 = (px2 - px1) * (py2 - py1)
    iou = inter / (a1 + a2 - inter + 1e-9)
    is_in = jnp.minimum(jnp.minimum(ax - gx1, ay - gy1),
                        jnp.minimum(gx2 - ax, gy2 - ay)) > 0.01
    iota_c = jax.lax.broadcasted_iota(jnp.int32, (1, c), 1)
    onehot = (cls_ref[...] == iota_c).astype(jnp.float32)
    pps = jax.lax.dot_general(
        onehot, scores_ref[...], (((1,), (1,)), ((), ())),
        precision=jax.lax.Precision.HIGHEST,
        preferred_element_type=jnp.float32)
    cost = (_softplus(-pps) - _IOU_WEIGHT * jnp.log(iou + 1e-8)
            + 1e5 * (~is_in).astype(jnp.float32))
    key = _sortable_key(cost)
    iou_m = jnp.where(is_in, iou, 0.0)

    k = jnp.clip(iou_m.sum(axis=1, keepdims=True).astype(jnp.int32), 1, a)

    def body_val(_, lh):
        lo, hi = lh
        mid = lo + jax.lax.shift_right_logical(hi - lo, 1)
        pred = _rowcount(key <= mid) >= k
        return (jnp.where(pred, lo, mid + 1), jnp.where(pred, mid, hi))

    t, _ = jax.lax.fori_loop(
        0, 32, body_val,
        (jnp.full((g, 1), _I32_MIN, jnp.int32),
         jnp.full((g, 1), _I32_MAX, jnp.int32)))
    n_lt = _rowcount(key < t)
    need = k - n_lt
    n_eq = _rowcount(key == t)

    def body_idx(_, lh):
        lo, hi = lh
        mid = lo + jax.lax.shift_right_logical(hi - lo, 1)
        idx_b = jax.lax.broadcasted_iota(jnp.int32, (g, a), 1)
        pred = _rowcount((key == t) & (idx_b <= mid)) >= need
        return (jnp.where(pred, lo, mid + 1), jnp.where(pred, mid, hi))

    def _tie_search(_):
        lo2, _ = jax.lax.fori_loop(
            0, 15, body_idx,
            (jnp.zeros((g, 1), jnp.int32),
             jnp.full((g, 1), a - 1, jnp.int32)))
        return lo2

    tie_i = jax.lax.cond(jnp.all(n_eq == need), lambda _: jnp.full(
        (g, 1), a - 1, jnp.int32), _tie_search, operand=0)

    idx = jax.lax.broadcasted_iota(jnp.int32, (g, a), 1)
    mm = (key < t) | ((key == t) & (idx <= tie_i))
    mmf = mm.astype(jnp.float32)
    amg = mmf.sum(axis=0, keepdims=True)
    conflict_f = (amg > 1.0).astype(jnp.float32)
    minkey = key.min(axis=0, keepdims=True)
    iota_g = jax.lax.broadcasted_iota(jnp.int32, (g, a), 0)
    cm = jnp.where(key == minkey, iota_g, jnp.int32(g)).min(
        axis=0, keepdims=True)
    onehot_f = (iota_g == cm).astype(jnp.float32)
    mmf2 = conflict_f * onehot_f + (1.0 - conflict_f) * mmf

    matched_iou = (mmf2 * iou_m).sum(axis=0, keepdims=True)
    dot_s = (mmf2 * iou_m * pps).sum()
    sp_s = _softplus(scores_ref[...]).sum()
    s1 = matched_iou.sum()

    bx1 = (mmf2 * gx1).sum(axis=0, keepdims=True)
    by1 = (mmf2 * gy1).sum(axis=0, keepdims=True)
    bx2 = (mmf2 * gx2).sum(axis=0, keepdims=True)
    by2 = (mmf2 * gy2).sum(axis=0, keepdims=True)
    eps = 1e-7
    w1 = px2 - px1
    h1 = py2 - py1 + eps
    w2 = bx2 - bx1
    h2 = by2 - by1 + eps
    inter2 = (jnp.clip(jnp.minimum(px2, bx2) - jnp.maximum(px1, bx1), 0, None)
              * jnp.clip(jnp.minimum(py2, by2) - jnp.maximum(py1, by1), 0, None))
    union2 = w1 * h1 + w2 * h2 - inter2 + eps
    iou2 = inter2 / union2
    cw = jnp.maximum(px2, bx2) - jnp.minimum(px1, bx1)
    ch = jnp.maximum(py2, by2) - jnp.minimum(py1, by1)
    c2 = cw * cw + ch * ch + eps
    rho2 = ((bx1 + bx2 - px1 - px2) ** 2 + (by1 + by2 - py1 - py2) ** 2) / 4.0
    v = (4.0 / math.pi**2) * (_arctan(w2 / h2) - _arctan(w1 / h1)) ** 2
    alpha = v / (v - iou2 + (1.0 + eps))
    ciou = iou2 - (rho2 / c2 + v * alpha)
    box_s = ((1.0 - ciou) * matched_iou).sum()

    tss = jnp.maximum(s1, 1.0)
    loss = (_LAMBDA_BOX * box_s + _LAMBDA_CLS * (sp_s - dot_s)) / tss
    loss_ref[...] = loss.reshape(1, 1)


def kernel(pd_scores, pd_bboxes, anc_points, gt_labels, gt_bboxes):
    cls2d = gt_labels[:, 0:1].astype(jnp.int32)
    out = pl.pallas_call(
        _fused,
        out_shape=jax.ShapeDtypeStruct((1, 1), jnp.float32),
    )(pd_scores, pd_bboxes.T, anc_points.T, gt_bboxes, cls2d)
    return out[0, 0]
